# baseline (speedup 1.0000x reference)
import jax
import jax.numpy as jnp
from jax import lax
from jax.experimental import pallas as pl
from jax.experimental.pallas import tpu as pltpu
from jax.experimental.pallas import tpu_sc as plsc

_K = 100000
_B = 16384
_NS = 50
_N = _B * _NS
_NW = 32
_NPW = _N // _NW
_L = 16
_NCHUNK = 16
_C = _NPW // _NCHUNK
_RC = _C // _NS
_ST = 6400
_FCAP = 128


def _body(prob_hbm, alias_hbm, kk_hbm, u_hbm, out_hbm, scr_hbm,
          packed_t, kk0, kk1, u0, u1, o0, o1,
          fkk, fu, frow, fcol, fp,
          sk0, sk1, su0, su1, sg0, so0, so1):
    c = lax.axis_index("c")
    s = lax.axis_index("s")
    wid = s * 2 + c
    base = wid * _NPW
    rbase = wid * (_NPW // _NS)

    kk_v = (kk0, kk1)
    u_v = (u0, u1)
    o_v = (o0, o1)
    sk = (sk0, sk1)
    su = (su0, su1)
    so = (so0, so1)

    def in_copies(i):
        off = base + i * _C
        j = i % 2
        ck = pltpu.async_copy(kk_hbm.at[pl.ds(off, _C)], kk_v[j], sk[j])
        cu = pltpu.async_copy(u_hbm.at[pl.ds(off, _C)], u_v[j], su[j])
        return ck, cu

    ck, cu = in_copies(0)

    def zstep(t, carry):
        fkk[pl.ds(t * _L, _L)] = jnp.zeros((_L,), jnp.int32)
        return carry

    lax.fori_loop(0, _FCAP // _L, zstep, 0)

    with jax.named_scope("pack_build"):
        o0_ = jnp.where(s < 15, s * _ST, _K - 4000)
        o1_ = jnp.where(s < 15, s * _ST + 3200, _K - 3200)
        for off in (o0_, o0_ + 1600, o1_, o1_ + 1600):
            pltpu.sync_copy(alias_hbm.at[pl.ds(off, _C)], kk1)
            pltpu.sync_copy(prob_hbm.at[pl.ds(off, _C)], u1)

            def pstep(t, carry):
                sl = pl.ds(t * _L, _L)
                ps = u1[sl] * 32768.0
                ti = ps.astype(jnp.int32)
                tc = ti + jnp.where(ti.astype(jnp.float32) < ps, 1, 0)
                t15 = jnp.minimum(tc, 32767)
                kk1[sl] = jnp.bitwise_or(jnp.left_shift(t15, 17), kk1[sl])
                return carry

            lax.fori_loop(0, _C // _L, pstep, 0)
            pltpu.sync_copy(kk1, scr_hbm.at[pl.ds(c * _K + off, _C)])
        plsc.subcore_barrier()
    with jax.named_scope("pack_pull"):
        pltpu.sync_copy(scr_hbm.at[pl.ds(c * _K, _K)], packed_t)

    ck.wait()
    cu.wait()
    ck, cu = in_copies(1)

    out_cp = [None, None]
    for i in range(_NCHUNK):
        j = i % 2
        with jax.named_scope(f"hdr{i}"):
            if out_cp[j] is not None:
                out_cp[j].wait()

        with jax.named_scope(f"sel{i}"):
            def step(t, carry, j=j):
                row, col, cnt = carry
                sl = pl.ds(t * _L, _L)
                kkv = kk_v[j][sl]
                w = plsc.bitcast(plsc.load_gather(packed_t, [kkv]),
                                 jnp.uint32)
                k15 = (u_v[j][sl] * 32768.0).astype(jnp.int32)
                t15 = (w >> 17).astype(jnp.int32)
                av = (w & 0x1FFFF).astype(jnp.int32)
                val = jnp.where(k15 < t15, kkv, av)
                plsc.store_scatter(o_v[j], [row, col], val)
                ambig = (k15 >= t15 - 1) & (k15 <= t15)
                cnt = cnt + plsc.all_reduce_population_count(ambig)
                col = col + _L
                wrap = col >= _NS
                col = jnp.where(wrap, col - _NS, col)
                row = row + wrap.astype(jnp.int32)
                return row, col, cnt

            _, _, cnt_v = lax.fori_loop(
                0, _C // _L, step,
                (jnp.zeros((_L,), jnp.int32), lax.iota(jnp.int32, _L),
                 jnp.zeros((_L,), jnp.int32)))
            cnt = jnp.max(cnt_v)

        @pl.when(cnt > 0)
        def _fixup(i=i, j=j):
            with jax.named_scope(f"fix{i}"):
                def cap(t, carry, j=j):
                    row, col, cc = carry
                    sl = pl.ds(t * _L, _L)
                    kkv = kk_v[j][sl]
                    w = plsc.bitcast(plsc.load_gather(packed_t, [kkv]),
                                     jnp.uint32)
                    k15 = (u_v[j][sl] * 32768.0).astype(jnp.int32)
                    t15 = (w >> 17).astype(jnp.int32)
                    m = (k15 >= t15 - 1) & (k15 <= t15)
                    dst = pl.ds(jnp.minimum(cc, _FCAP - _L), _L)
                    plsc.store_compressed(fkk.at[dst], kkv, mask=m)
                    plsc.store_compressed(fu.at[dst], u_v[j][sl], mask=m)
                    plsc.store_compressed(frow.at[dst], row, mask=m)
                    plsc.store_compressed(fcol.at[dst], col, mask=m)
                    cc = cc + jnp.max(plsc.all_reduce_population_count(m))
                    col = col + _L
                    wrap = col >= _NS
                    col = jnp.where(wrap, col - _NS, col)
                    row = row + wrap.astype(jnp.int32)
                    return row, col, cc

                _, _, nfix = lax.fori_loop(
                    0, _C // _L, cap,
                    (jnp.zeros((_L,), jnp.int32), lax.iota(jnp.int32, _L),
                     jnp.int32(0)))
                nfix = jnp.minimum(nfix, _FCAP - _L)
                pltpu.async_copy(prob_hbm.at[fkk], fp, sg0).wait()

                def fstep(t, carry, j=j):
                    sl = pl.ds(t * _L, _L)
                    kkv = fkk[sl]
                    b = fu[sl] < fp[sl]
                    w = plsc.bitcast(plsc.load_gather(packed_t, [kkv]),
                                     jnp.uint32)
                    av = (w & 0x1FFFF).astype(jnp.int32)
                    val = jnp.where(b, kkv, av)
                    lane = t * _L + lax.iota(jnp.int32, _L)
                    plsc.store_scatter(o_v[j], [frow[sl], fcol[sl]], val,
                                       mask=lane < nfix)
                    return carry

                lax.fori_loop(0, _FCAP // _L, fstep, 0)

        out_cp[j] = pltpu.async_copy(
            o_v[j], out_hbm.at[pl.ds(rbase + i * _RC, _RC), :], so[j])
        if i + 1 < _NCHUNK:
            ck.wait()
            cu.wait()
        if i + 2 < _NCHUNK:
            ck, cu = in_copies(i + 2)

    out_cp[0].wait()
    out_cp[1].wait()


@jax.jit
def _sample(prob, alias, kk, u):
    mesh = plsc.VectorSubcoreMesh(core_axis_name="c", subcore_axis_name="s")
    f = pl.kernel(
        _body,
        mesh=mesh,
        compiler_params=pltpu.CompilerParams(needs_layout_passes=False),
        out_type=(jax.ShapeDtypeStruct((_B, 128), jnp.int32),
                  jax.ShapeDtypeStruct((2 * _K,), jnp.int32)),
        scratch_types=[
            pltpu.VMEM((_K,), jnp.int32),
            pltpu.VMEM((_C,), jnp.int32),
            pltpu.VMEM((_C,), jnp.int32),
            pltpu.VMEM((_C,), jnp.float32),
            pltpu.VMEM((_C,), jnp.float32),
            pltpu.VMEM((_RC, 128), jnp.int32),
            pltpu.VMEM((_RC, 128), jnp.int32),
            pltpu.VMEM((_FCAP,), jnp.int32),
            pltpu.VMEM((_FCAP,), jnp.float32),
            pltpu.VMEM((_FCAP,), jnp.int32),
            pltpu.VMEM((_FCAP,), jnp.int32),
            pltpu.VMEM((_FCAP,), jnp.float32),
        ] + [pltpu.SemaphoreType.DMA] * 7,
    )
    out, _ = f(prob, alias, kk, u)
    return out


def kernel(prob, alias, kk, u):
    return _sample(prob, alias, kk, u)[:, :_NS]


# speedup vs baseline: 1.4194x; 1.4194x over previous
import jax
import jax.numpy as jnp
from jax import lax
from jax.experimental import pallas as pl
from jax.experimental.pallas import tpu as pltpu
from jax.experimental.pallas import tpu_sc as plsc

_K = 100000
_B = 16384
_NS = 50
_N = _B * _NS
_NW = 32
_NPW = _N // _NW
_L = 16
_NCHUNK = 16
_C = _NPW // _NCHUNK
_RC = _C // _NS
_ST = 6400
_FCAP = 128


def _unpack(w):
    t15 = (w >> 17).astype(jnp.int32)
    av = (w & 0x1FFFF).astype(jnp.int32)
    return t15, av


def _ambig(k15, t15):
    d = plsc.bitcast(k15 - t15 + 1, jnp.uint32)
    return d <= jnp.uint32(1)


def _body(prob_hbm, alias_hbm, kk_hbm, u_hbm, out_hbm, scr_hbm,
          packed_t, prob_s, kk0, kk1, u0, u1, o0, o1, fpos, fkk, fp,
          sk0, sk1, su0, su1, sg0, so0, so1):
    c = lax.axis_index("c")
    s = lax.axis_index("s")
    wid = s * 2 + c
    base = wid * _NPW
    rbase = wid * (_NPW // _NS)

    kk_v = (kk0, kk1)
    u_v = (u0, u1)
    o_v = (o0, o1)
    sk = (sk0, sk1)
    su = (su0, su1)
    so = (so0, so1)

    def in_copies(i):
        off = base + i * _C
        j = i % 2
        ck = pltpu.async_copy(kk_hbm.at[pl.ds(off, _C)], kk_v[j], sk[j])
        cu = pltpu.async_copy(u_hbm.at[pl.ds(off, _C)], u_v[j], su[j])
        return ck, cu

    def zstep(t, carry):
        fpos[pl.ds(t * _L, _L)] = jnp.zeros((_L,), jnp.int32)
        return carry

    lax.fori_loop(0, _FCAP // _L, zstep, 0)

    with jax.named_scope("pack_build"):
        o0_ = jnp.where(s < 15, s * _ST, _K - 4000)
        o1_ = jnp.where(s < 15, s * _ST + 3200, _K - 3200)
        offs = (o0_, o0_ + 1600, o1_, o1_ + 1600)

        def stripe_reads(t2):
            j = t2 % 2
            ca = pltpu.async_copy(alias_hbm.at[pl.ds(offs[t2], _C)],
                                  kk_v[j], sk[j])
            cp_ = pltpu.async_copy(prob_hbm.at[pl.ds(offs[t2], _C)],
                                   u_v[j], su[j])
            return ca, cp_

        rd = stripe_reads(0)
        wr = [None, None]
        for t2 in range(4):
            j = t2 % 2
            nj = 1 - j
            rd[0].wait()
            rd[1].wait()
            if t2 + 1 < 4:
                if wr[nj] is not None:
                    wr[nj][0].wait()
                    wr[nj][1].wait()
                    wr[nj] = None
                rd_next = stripe_reads(t2 + 1)

            def pstep(t, carry, j=j):
                sl = pl.ds(t * _L, _L)
                ps = u_v[j][sl] * 32768.0
                ti = ps.astype(jnp.int32)
                tc = ti + jnp.where(ti.astype(jnp.float32) < ps, 1, 0)
                t15 = jnp.minimum(tc, 32767)
                kk_v[j][sl] = jnp.bitwise_or(jnp.left_shift(t15, 17),
                                             kk_v[j][sl])
                return carry

            plsc.parallel_loop(0, _C // _L, unroll=2,
                               carry=jnp.int32(0))(pstep)
            wr[j] = (
                pltpu.async_copy(kk_v[j],
                                 scr_hbm.at[pl.ds(c * _K + offs[t2], _C)],
                                 so[j]),
                pltpu.async_copy(u_v[j], prob_s.at[pl.ds(offs[t2], _C)],
                                 sg0),
            )
            if t2 + 1 < 4:
                rd = rd_next
        for w2 in wr:
            if w2 is not None:
                w2[0].wait()
                w2[1].wait()
        plsc.subcore_barrier()
    with jax.named_scope("pack_pull"):
        pltpu.sync_copy(scr_hbm.at[pl.ds(c * _K, _K)], packed_t)

    ck, cu = in_copies(0)
    ck.wait()
    cu.wait()
    ck, cu = in_copies(1)

    out_cp = [None, None]
    for i in range(_NCHUNK):
        j = i % 2
        if out_cp[j] is not None:
            out_cp[j].wait()

        with jax.named_scope(f"sel{i}"):
            def step(t, carry, j=j):
                row, col, cnt = carry
                sl = pl.ds(t * _L, _L)
                kkv = kk_v[j][sl]
                w = plsc.bitcast(plsc.load_gather(packed_t, [kkv]),
                                 jnp.uint32)
                k15 = (u_v[j][sl] * 32768.0).astype(jnp.int32)
                t15, av = _unpack(w)
                val = jnp.where(k15 < t15, kkv, av)
                plsc.store_scatter(o_v[j], [row, col], val)
                cnt = cnt + plsc.all_reduce_population_count(
                    _ambig(k15, t15))
                col = col + _L
                wrap = col >= _NS
                col = jnp.where(wrap, col - _NS, col)
                row = row + wrap.astype(jnp.int32)
                return row, col, cnt

            init = (jnp.zeros((_L,), jnp.int32), lax.iota(jnp.int32, _L),
                    jnp.zeros((_L,), jnp.int32))
            _, _, cnt_v = plsc.parallel_loop(
                0, _C // _L, unroll=4, carry=init)(step)
            cnt = jnp.max(cnt_v)

        @pl.when(cnt > 0)
        def _fixup(i=i, j=j):
            with jax.named_scope(f"fix{i}"):
                def cap(t, cc, j=j):
                    sl = pl.ds(t * _L, _L)
                    kkv = kk_v[j][sl]
                    w = plsc.bitcast(plsc.load_gather(packed_t, [kkv]),
                                     jnp.uint32)
                    k15 = (u_v[j][sl] * 32768.0).astype(jnp.int32)
                    t15, _ = _unpack(w)
                    m = _ambig(k15, t15)
                    dst = pl.ds(jnp.minimum(cc, _FCAP - _L), _L)
                    plsc.store_compressed(
                        fpos.at[dst], t * _L + lax.iota(jnp.int32, _L),
                        mask=m)
                    return cc + jnp.max(
                        plsc.all_reduce_population_count(m))

                nfix = lax.fori_loop(0, _C // _L, cap, jnp.int32(0))
                nfix = jnp.minimum(nfix, _FCAP - _L)

                def istep(t, carry, j=j):
                    sl = pl.ds(t * _L, _L)
                    fkk[sl] = plsc.load_gather(kk_v[j], [fpos[sl]])
                    return carry

                lax.fori_loop(0, _FCAP // _L, istep, 0)
                pltpu.async_copy(prob_s.at[fkk], fp, sg0).wait()

                def fstep(t, carry, j=j):
                    sl = pl.ds(t * _L, _L)
                    pos = fpos[sl]
                    kkv = fkk[sl]
                    uv = plsc.load_gather(u_v[j], [pos])
                    b = uv < fp[sl]
                    w = plsc.bitcast(plsc.load_gather(packed_t, [kkv]),
                                     jnp.uint32)
                    _, av = _unpack(w)
                    val = jnp.where(b, kkv, av)
                    row = (pos * 5243) >> 18
                    colx = pos - row * _NS
                    lane = t * _L + lax.iota(jnp.int32, _L)
                    plsc.store_scatter(o_v[j], [row, colx], val,
                                       mask=lane < nfix)
                    return carry

                lax.fori_loop(0, _FCAP // _L, fstep, 0)

        out_cp[j] = pltpu.async_copy(
            o_v[j], out_hbm.at[pl.ds(rbase + i * _RC, _RC), :], so[j])
        if i + 1 < _NCHUNK:
            ck.wait()
            cu.wait()
        if i + 2 < _NCHUNK:
            ck, cu = in_copies(i + 2)

    out_cp[0].wait()
    out_cp[1].wait()


@jax.jit
def _sample(prob, alias, kk, u):
    mesh = plsc.VectorSubcoreMesh(core_axis_name="c", subcore_axis_name="s")
    f = pl.kernel(
        _body,
        mesh=mesh,
        compiler_params=pltpu.CompilerParams(needs_layout_passes=False),
        out_type=(jax.ShapeDtypeStruct((_B, 128), jnp.int32),
                  jax.ShapeDtypeStruct((2 * _K,), jnp.int32)),
        scratch_types=[
            pltpu.VMEM((_K,), jnp.int32),
            pltpu.VMEM_SHARED((_K,), jnp.float32),
            pltpu.VMEM((_C,), jnp.int32),
            pltpu.VMEM((_C,), jnp.int32),
            pltpu.VMEM((_C,), jnp.float32),
            pltpu.VMEM((_C,), jnp.float32),
            pltpu.VMEM((_RC, 128), jnp.int32),
            pltpu.VMEM((_RC, 128), jnp.int32),
            pltpu.VMEM((_FCAP,), jnp.int32),
            pltpu.VMEM((_FCAP,), jnp.int32),
            pltpu.VMEM((_FCAP,), jnp.float32),
        ] + [pltpu.SemaphoreType.DMA] * 7,
    )
    out, _ = f(prob, alias, kk, u)
    return out


def kernel(prob, alias, kk, u):
    return _sample(prob, alias, kk, u)[:, :_NS]


# speedup vs baseline: 1.6035x; 1.1297x over previous
import jax
import jax.numpy as jnp
from jax import lax
from jax.experimental import pallas as pl
from jax.experimental.pallas import tpu as pltpu
from jax.experimental.pallas import tpu_sc as plsc

_K = 100000
_B = 16384
_NS = 50
_N = _B * _NS
_NW = 32
_NPW = _N // _NW
_L = 16
_NCHUNK = 16
_C = _NPW // _NCHUNK
_RC = _C // _NS
_ST = 6400
_FCAP = 128


def _unpack(w):
    t15 = (w >> 17).astype(jnp.int32)
    av = (w & 0x1FFFF).astype(jnp.int32)
    return t15, av


def _ambig(k15, t15):
    d = plsc.bitcast(k15 - t15 + 1, jnp.uint32)
    return d <= jnp.uint32(1)


def _body(prob_hbm, alias_hbm, kk_hbm, u_hbm, out_hbm, scr_hbm,
          packed_t, prob_s, kk0, kk1, u0, u1, o0, o1, fpos, fkk, fp,
          sk0, sk1, su0, su1, sg0, so0, so1):
    c = lax.axis_index("c")
    s = lax.axis_index("s")
    wid = s * 2 + c
    base = wid * _NPW
    rbase = wid * (_NPW // _NS)

    kk_v = (kk0, kk1)
    u_v = (u0, u1)
    o_v = (o0, o1)
    sk = (sk0, sk1)
    su = (su0, su1)
    so = (so0, so1)

    def in_copies(i, j):
        off = base + i * _C
        ck = pltpu.async_copy(kk_hbm.at[pl.ds(off, _C)], kk_v[j], sk[j])
        cu = pltpu.async_copy(u_hbm.at[pl.ds(off, _C)], u_v[j], su[j])
        return ck, cu

    def zstep(t, carry):
        fpos[pl.ds(t * _L, _L)] = jnp.zeros((_L,), jnp.int32)
        return carry

    lax.fori_loop(0, _FCAP // _L, zstep, 0)

    with jax.named_scope("pack_build"):
        o0_ = jnp.where(s < 15, s * _ST, _K - 4000)
        o1_ = jnp.where(s < 15, s * _ST + 3200, _K - 3200)
        offs = (o0_, o0_ + 1600, o1_, o1_ + 1600)

        def stripe_reads(t2):
            j = t2 % 2
            ca = pltpu.async_copy(alias_hbm.at[pl.ds(offs[t2], _C)],
                                  kk_v[j], sk[j])
            cp_ = pltpu.async_copy(prob_hbm.at[pl.ds(offs[t2], _C)],
                                   u_v[j], su[j])
            return ca, cp_

        rd = stripe_reads(0)
        wr = [None, None]
        for t2 in range(4):
            j = t2 % 2
            nj = 1 - j
            rd[0].wait()
            rd[1].wait()
            if t2 + 1 < 4:
                if wr[nj] is not None:
                    wr[nj][0].wait()
                    wr[nj][1].wait()
                    wr[nj] = None
                rd_next = stripe_reads(t2 + 1)

            def pstep(t, carry, j=j):
                sl = pl.ds(t * _L, _L)
                ps = u_v[j][sl] * 32768.0
                ti = ps.astype(jnp.int32)
                tc = ti + jnp.where(ti.astype(jnp.float32) < ps, 1, 0)
                t15 = jnp.minimum(tc, 32767)
                kk_v[j][sl] = jnp.bitwise_or(jnp.left_shift(t15, 17),
                                             kk_v[j][sl])
                return carry

            plsc.parallel_loop(0, _C // _L, unroll=2,
                               carry=jnp.int32(0))(pstep)
            wr[j] = (
                pltpu.async_copy(kk_v[j],
                                 scr_hbm.at[pl.ds(c * _K + offs[t2], _C)],
                                 so[j]),
                pltpu.async_copy(u_v[j], prob_s.at[pl.ds(offs[t2], _C)],
                                 sg0),
            )
            if t2 + 1 < 4:
                rd = rd_next
        for w2 in wr:
            if w2 is not None:
                w2[0].wait()
                w2[1].wait()
        plsc.subcore_barrier()
    with jax.named_scope("pack_pull"):
        pltpu.sync_copy(scr_hbm.at[pl.ds(c * _K, _K)], packed_t)

    in_copies(0, 0)
    in_copies(1, 1)

    def chunk_pair(i2, carry):
      for j in (0, 1):
        i = i2 * 2 + j
        off = base + i * _C

        @pl.when(i2 >= 1)
        def _wait_prev(i=i, j=j):
            pltpu.make_async_copy(
                o_v[j],
                out_hbm.at[pl.ds(rbase + (i - 2) * _RC, _RC), :],
                so[j]).wait()

        pltpu.make_async_copy(
            kk_hbm.at[pl.ds(off, _C)], kk_v[j], sk[j]).wait()
        pltpu.make_async_copy(
            u_hbm.at[pl.ds(off, _C)], u_v[j], su[j]).wait()

        with jax.named_scope("sel"):
            def step(t, carry, j=j):
                row, col, cnt = carry
                sl = pl.ds(t * _L, _L)
                kkv = kk_v[j][sl]
                w = plsc.bitcast(plsc.load_gather(packed_t, [kkv]),
                                 jnp.uint32)
                k15 = (u_v[j][sl] * 32768.0).astype(jnp.int32)
                t15, av = _unpack(w)
                val = jnp.where(k15 < t15, kkv, av)
                plsc.store_scatter(o_v[j], [row, col], val)
                cnt = cnt + plsc.all_reduce_population_count(
                    _ambig(k15, t15))
                col = col + _L
                wrap = col >= _NS
                col = jnp.where(wrap, col - _NS, col)
                row = row + wrap.astype(jnp.int32)
                return row, col, cnt

            init = (jnp.zeros((_L,), jnp.int32), lax.iota(jnp.int32, _L),
                    jnp.zeros((_L,), jnp.int32))
            _, _, cnt_v = plsc.parallel_loop(
                0, _C // _L, unroll=4, carry=init)(step)
            cnt = jnp.max(cnt_v)

        @pl.when(cnt > 0)
        def _fixup(i=i, j=j):
            with jax.named_scope("fix"):
                def cap(t, cc, j=j):
                    sl = pl.ds(t * _L, _L)
                    kkv = kk_v[j][sl]
                    w = plsc.bitcast(plsc.load_gather(packed_t, [kkv]),
                                     jnp.uint32)
                    k15 = (u_v[j][sl] * 32768.0).astype(jnp.int32)
                    t15, _ = _unpack(w)
                    m = _ambig(k15, t15)
                    dst = pl.ds(jnp.minimum(cc, _FCAP - _L), _L)
                    plsc.store_compressed(
                        fpos.at[dst], t * _L + lax.iota(jnp.int32, _L),
                        mask=m)
                    return cc + jnp.max(
                        plsc.all_reduce_population_count(m))

                nfix = lax.fori_loop(0, _C // _L, cap, jnp.int32(0))
                nfix = jnp.minimum(nfix, _FCAP - _L)

                def istep(t, carry, j=j):
                    sl = pl.ds(t * _L, _L)
                    fkk[sl] = plsc.load_gather(kk_v[j], [fpos[sl]])
                    return carry

                lax.fori_loop(0, _FCAP // _L, istep, 0)
                pltpu.async_copy(prob_s.at[fkk], fp, sg0).wait()

                def fstep(t, carry, j=j):
                    sl = pl.ds(t * _L, _L)
                    pos = fpos[sl]
                    kkv = fkk[sl]
                    uv = plsc.load_gather(u_v[j], [pos])
                    b = uv < fp[sl]
                    w = plsc.bitcast(plsc.load_gather(packed_t, [kkv]),
                                     jnp.uint32)
                    _, av = _unpack(w)
                    val = jnp.where(b, kkv, av)
                    row = (pos * 5243) >> 18
                    colx = pos - row * _NS
                    lane = t * _L + lax.iota(jnp.int32, _L)
                    plsc.store_scatter(o_v[j], [row, colx], val,
                                       mask=lane < nfix)
                    return carry

                lax.fori_loop(0, _FCAP // _L, fstep, 0)

        pltpu.async_copy(
            o_v[j], out_hbm.at[pl.ds(rbase + i * _RC, _RC), :], so[j])

        @pl.when(i2 < (_NCHUNK // 2) - 1)
        def _next_in(i=i, j=j):
            in_copies(i + 2, j)
      return carry

    lax.fori_loop(0, _NCHUNK // 2, chunk_pair, 0)

    pltpu.make_async_copy(
        o_v[0], out_hbm.at[pl.ds(rbase + 14 * _RC, _RC), :], so[0]).wait()
    pltpu.make_async_copy(
        o_v[1], out_hbm.at[pl.ds(rbase + 15 * _RC, _RC), :], so[1]).wait()


@jax.jit
def _sample(prob, alias, kk, u):
    mesh = plsc.VectorSubcoreMesh(core_axis_name="c", subcore_axis_name="s")
    f = pl.kernel(
        _body,
        mesh=mesh,
        compiler_params=pltpu.CompilerParams(needs_layout_passes=False),
        out_type=(jax.ShapeDtypeStruct((_B, 128), jnp.int32),
                  jax.ShapeDtypeStruct((2 * _K,), jnp.int32)),
        scratch_types=[
            pltpu.VMEM((_K,), jnp.int32),
            pltpu.VMEM_SHARED((_K,), jnp.float32),
            pltpu.VMEM((_C,), jnp.int32),
            pltpu.VMEM((_C,), jnp.int32),
            pltpu.VMEM((_C,), jnp.float32),
            pltpu.VMEM((_C,), jnp.float32),
            pltpu.VMEM((_RC, 128), jnp.int32),
            pltpu.VMEM((_RC, 128), jnp.int32),
            pltpu.VMEM((_FCAP,), jnp.int32),
            pltpu.VMEM((_FCAP,), jnp.int32),
            pltpu.VMEM((_FCAP,), jnp.float32),
        ] + [pltpu.SemaphoreType.DMA] * 7,
    )
    out, _ = f(prob, alias, kk, u)
    return out


def kernel(prob, alias, kk, u):
    return _sample(prob, alias, kk, u)[:, :_NS]


# speedup vs baseline: 1.7720x; 1.1051x over previous
import jax
import jax.numpy as jnp
from jax import lax
from jax.experimental import pallas as pl
from jax.experimental.pallas import tpu as pltpu
from jax.experimental.pallas import tpu_sc as plsc

_K = 100000
_B = 16384
_NS = 50
_N = _B * _NS
_NW = 32
_NPW = _N // _NW
_L = 16
_NCHUNK = 16
_C = _NPW // _NCHUNK
_RC = _C // _NS
_ST = 6400
_FCAP = 128


def _unpack(w):
    t15 = (w >> 17).astype(jnp.int32)
    av = (w & 0x1FFFF).astype(jnp.int32)
    return t15, av


def _ambig(k15, t15):
    d = plsc.bitcast(k15 - t15 + 1, jnp.uint32)
    return d <= jnp.uint32(1)


def _body(prob_hbm, alias_hbm, kk_hbm, u_hbm, out_hbm, scr_hbm,
          packed_t, prob_s, kk0, kk1, u0, u1, o0, o1, fpos, fkk, fp,
          sk0, sk1, su0, su1, sg0, so0, so1, sp0, sp1):
    c = lax.axis_index("c")
    s = lax.axis_index("s")
    wid = s * 2 + c
    base = wid * _NPW
    rbase = wid * (_NPW // _NS)

    kk_v = (kk0, kk1)
    u_v = (u0, u1)
    o_v = (o0, o1)
    sk = (sk0, sk1)
    su = (su0, su1)
    so = (so0, so1)

    def in_copies(i, j):
        off = base + i * _C
        ck = pltpu.async_copy(kk_hbm.at[pl.ds(off, _C)], kk_v[j], sk[j])
        cu = pltpu.async_copy(u_hbm.at[pl.ds(off, _C)], u_v[j], su[j])
        return ck, cu

    def zstep(t, carry):
        fpos[pl.ds(t * _L, _L)] = jnp.zeros((_L,), jnp.int32)
        return carry

    lax.fori_loop(0, _FCAP // _L, zstep, 0)

    with jax.named_scope("pack_build"):
        o0_ = jnp.where(s < 15, s * _ST, _K - 4000)
        o1_ = jnp.where(s < 15, s * _ST + 3200, _K - 3200)
        offs = (o0_, o0_ + 1600, o1_, o1_ + 1600)

        def stripe_reads(t2):
            j = t2 % 2
            ca = pltpu.async_copy(alias_hbm.at[pl.ds(offs[t2], _C)],
                                  kk_v[j], sk[j])
            cp_ = pltpu.async_copy(prob_hbm.at[pl.ds(offs[t2], _C)],
                                   u_v[j], su[j])
            return ca, cp_

        rd = stripe_reads(0)
        wr = [None, None]
        for t2 in range(4):
            j = t2 % 2
            nj = 1 - j
            rd[0].wait()
            rd[1].wait()
            if t2 + 1 < 4:
                if wr[nj] is not None:
                    wr[nj][0].wait()
                    wr[nj][1].wait()
                    wr[nj] = None
                rd_next = stripe_reads(t2 + 1)

            def pstep(t, carry, j=j):
                sl = pl.ds(t * _L, _L)
                ps = u_v[j][sl] * 32768.0
                ti = ps.astype(jnp.int32)
                tc = ti + jnp.where(ti.astype(jnp.float32) < ps, 1, 0)
                t15 = jnp.minimum(tc, 32767)
                kk_v[j][sl] = jnp.bitwise_or(jnp.left_shift(t15, 17),
                                             kk_v[j][sl])
                return carry

            plsc.parallel_loop(0, _C // _L, unroll=2,
                               carry=jnp.int32(0))(pstep)
            wr[j] = (
                pltpu.async_copy(kk_v[j],
                                 scr_hbm.at[pl.ds(c * _K + offs[t2], _C)],
                                 so[j]),
                pltpu.async_copy(u_v[j], prob_s.at[pl.ds(offs[t2], _C)],
                                 sg0),
            )
            if t2 + 1 < 4:
                rd = rd_next
        for w2 in wr:
            if w2 is not None:
                w2[0].wait()
                w2[1].wait()
        plsc.subcore_barrier()
    in_copies(0, 0)
    in_copies(1, 1)
    with jax.named_scope("pack_pull"):
        _H = _K // 2
        p1 = pltpu.async_copy(scr_hbm.at[pl.ds(c * _K, _H)],
                              packed_t.at[pl.ds(0, _H)], sp0)
        p2 = pltpu.async_copy(scr_hbm.at[pl.ds(c * _K + _H, _H)],
                              packed_t.at[pl.ds(_H, _H)], sp1)
        p1.wait()
        p2.wait()

    def chunk_pair(i2, carry):
      for j in (0, 1):
        i = i2 * 2 + j
        off = base + i * _C

        @pl.when(i2 >= 1)
        def _wait_prev(i=i, j=j):
            pltpu.make_async_copy(
                o_v[j],
                out_hbm.at[pl.ds(rbase + (i - 2) * _RC, _RC), :],
                so[j]).wait()

        pltpu.make_async_copy(
            kk_hbm.at[pl.ds(off, _C)], kk_v[j], sk[j]).wait()
        pltpu.make_async_copy(
            u_hbm.at[pl.ds(off, _C)], u_v[j], su[j]).wait()

        with jax.named_scope("sel"):
            def step(t, carry, j=j):
                row, col, cnt, lo, hi = carry
                sl = pl.ds(t * _L, _L)
                kkv = kk_v[j][sl]
                w = plsc.bitcast(plsc.load_gather(packed_t, [kkv]),
                                 jnp.uint32)
                k15 = (u_v[j][sl] * 32768.0).astype(jnp.int32)
                t15, av = _unpack(w)
                val = jnp.where(k15 < t15, kkv, av)
                plsc.store_scatter(o_v[j], [row, col], val)
                pc = plsc.all_reduce_population_count(_ambig(k15, t15))
                cnt = cnt + pc
                lo = jnp.minimum(lo, jnp.where(pc > 0, t, _C))
                hi = jnp.maximum(hi, jnp.where(pc > 0, t + 1, 0))
                col = col + _L
                wrap = col >= _NS
                col = jnp.where(wrap, col - _NS, col)
                row = row + wrap.astype(jnp.int32)
                return row, col, cnt, lo, hi

            init = (jnp.zeros((_L,), jnp.int32), lax.iota(jnp.int32, _L),
                    jnp.zeros((_L,), jnp.int32),
                    jnp.full((_L,), _C, jnp.int32),
                    jnp.zeros((_L,), jnp.int32))
            _, _, cnt_v, lo_v, hi_v = plsc.parallel_loop(
                0, _C // _L, unroll=4, carry=init)(step)
            cnt = jnp.max(cnt_v)

        @pl.when(cnt > 0)
        def _fixup(i=i, j=j, lo_v=lo_v, hi_v=hi_v):
            with jax.named_scope("fix"):
                def cap(t, cc, j=j):
                    sl = pl.ds(t * _L, _L)
                    kkv = kk_v[j][sl]
                    w = plsc.bitcast(plsc.load_gather(packed_t, [kkv]),
                                     jnp.uint32)
                    k15 = (u_v[j][sl] * 32768.0).astype(jnp.int32)
                    t15, _ = _unpack(w)
                    m = _ambig(k15, t15)
                    dst = pl.ds(jnp.minimum(cc, _FCAP - _L), _L)
                    plsc.store_compressed(
                        fpos.at[dst], t * _L + lax.iota(jnp.int32, _L),
                        mask=m)
                    return cc + jnp.max(
                        plsc.all_reduce_population_count(m))

                nfix = lax.fori_loop(jnp.max(lo_v), jnp.max(hi_v), cap,
                                     jnp.int32(0))
                nfix = jnp.minimum(nfix, _FCAP - _L)

                def istep(t, carry, j=j):
                    sl = pl.ds(t * _L, _L)
                    fkk[sl] = plsc.load_gather(kk_v[j], [fpos[sl]])
                    return carry

                lax.fori_loop(0, _FCAP // _L, istep, 0)
                pltpu.async_copy(prob_s.at[fkk], fp, sg0).wait()

                def fstep(t, carry, j=j):
                    sl = pl.ds(t * _L, _L)
                    pos = fpos[sl]
                    kkv = fkk[sl]
                    uv = plsc.load_gather(u_v[j], [pos])
                    b = uv < fp[sl]
                    w = plsc.bitcast(plsc.load_gather(packed_t, [kkv]),
                                     jnp.uint32)
                    _, av = _unpack(w)
                    val = jnp.where(b, kkv, av)
                    row = (pos * 5243) >> 18
                    colx = pos - row * _NS
                    lane = t * _L + lax.iota(jnp.int32, _L)
                    plsc.store_scatter(o_v[j], [row, colx], val,
                                       mask=lane < nfix)
                    return carry

                lax.fori_loop(0, _FCAP // _L, fstep, 0)

        pltpu.async_copy(
            o_v[j], out_hbm.at[pl.ds(rbase + i * _RC, _RC), :], so[j])

        @pl.when(i2 < (_NCHUNK // 2) - 1)
        def _next_in(i=i, j=j):
            in_copies(i + 2, j)
      return carry

    lax.fori_loop(0, _NCHUNK // 2, chunk_pair, 0)

    pltpu.make_async_copy(
        o_v[0], out_hbm.at[pl.ds(rbase + 14 * _RC, _RC), :], so[0]).wait()
    pltpu.make_async_copy(
        o_v[1], out_hbm.at[pl.ds(rbase + 15 * _RC, _RC), :], so[1]).wait()


@jax.jit
def _sample(prob, alias, kk, u):
    mesh = plsc.VectorSubcoreMesh(core_axis_name="c", subcore_axis_name="s")
    f = pl.kernel(
        _body,
        mesh=mesh,
        compiler_params=pltpu.CompilerParams(needs_layout_passes=False),
        out_type=(jax.ShapeDtypeStruct((_B, 128), jnp.int32),
                  jax.ShapeDtypeStruct((2 * _K,), jnp.int32)),
        scratch_types=[
            pltpu.VMEM((_K,), jnp.int32),
            pltpu.VMEM_SHARED((_K,), jnp.float32),
            pltpu.VMEM((_C,), jnp.int32),
            pltpu.VMEM((_C,), jnp.int32),
            pltpu.VMEM((_C,), jnp.float32),
            pltpu.VMEM((_C,), jnp.float32),
            pltpu.VMEM((_RC, 128), jnp.int32),
            pltpu.VMEM((_RC, 128), jnp.int32),
            pltpu.VMEM((_FCAP,), jnp.int32),
            pltpu.VMEM((_FCAP,), jnp.int32),
            pltpu.VMEM((_FCAP,), jnp.float32),
        ] + [pltpu.SemaphoreType.DMA] * 9,
    )
    out, _ = f(prob, alias, kk, u)
    return out


def kernel(prob, alias, kk, u):
    return _sample(prob, alias, kk, u)[:, :_NS]


# speedup vs baseline: 1.7778x; 1.0033x over previous
import jax
import jax.numpy as jnp
from jax import lax
from jax.experimental import pallas as pl
from jax.experimental.pallas import tpu as pltpu
from jax.experimental.pallas import tpu_sc as plsc

_K = 100000
_B = 16384
_NS = 50
_N = _B * _NS
_NW = 32
_NPW = _N // _NW
_L = 16
_NCHUNK = 16
_C = _NPW // _NCHUNK
_RC = _C // _NS
_ST = 6400
_FCAP = 128
_OW = 64


def _unpack(w):
    t15 = (w >> 17).astype(jnp.int32)
    av = (w & 0x1FFFF).astype(jnp.int32)
    return t15, av


def _ambig(k15, t15):
    d = plsc.bitcast(k15 - t15 + 1, jnp.uint32)
    return d <= jnp.uint32(1)


def _body(prob_hbm, alias_hbm, kk_hbm, u_hbm, out_hbm, scr_hbm,
          packed_t, prob_s, kk0, kk1, u0, u1, o0, o1, fpos, fkk, fp,
          sk0, sk1, su0, su1, sg0, so0, so1, sp0, sp1):
    c = lax.axis_index("c")
    s = lax.axis_index("s")
    wid = s * 2 + c
    base = wid * _NPW
    rbase = wid * (_NPW // _NS)

    kk_v = (kk0, kk1)
    u_v = (u0, u1)
    o_v = (o0, o1)
    sk = (sk0, sk1)
    su = (su0, su1)
    so = (so0, so1)

    def in_copies(i, j):
        off = base + i * _C
        ck = pltpu.async_copy(kk_hbm.at[pl.ds(off, _C)], kk_v[j], sk[j])
        cu = pltpu.async_copy(u_hbm.at[pl.ds(off, _C)], u_v[j], su[j])
        return ck, cu

    def zstep(t, carry):
        fpos[pl.ds(t * _L, _L)] = jnp.zeros((_L,), jnp.int32)
        return carry

    lax.fori_loop(0, _FCAP // _L, zstep, 0)

    with jax.named_scope("pack_build"):
        o0_ = jnp.where(s < 15, s * _ST, _K - 4000)
        o1_ = jnp.where(s < 15, s * _ST + 3200, _K - 3200)
        offs = (o0_, o0_ + 1600, o1_, o1_ + 1600)

        def stripe_reads(t2):
            j = t2 % 2
            ca = pltpu.async_copy(alias_hbm.at[pl.ds(offs[t2], _C)],
                                  kk_v[j], sk[j])
            cp_ = pltpu.async_copy(prob_hbm.at[pl.ds(offs[t2], _C)],
                                   u_v[j], su[j])
            return ca, cp_

        rd = stripe_reads(0)
        wr = [None, None]
        for t2 in range(4):
            j = t2 % 2
            nj = 1 - j
            rd[0].wait()
            rd[1].wait()
            if t2 + 1 < 4:
                if wr[nj] is not None:
                    wr[nj][0].wait()
                    wr[nj][1].wait()
                    wr[nj] = None
                rd_next = stripe_reads(t2 + 1)

            def pstep(t, carry, j=j):
                sl = pl.ds(t * _L, _L)
                ps = u_v[j][sl] * 32768.0
                ti = ps.astype(jnp.int32)
                tc = ti + jnp.where(ti.astype(jnp.float32) < ps, 1, 0)
                t15 = jnp.minimum(tc, 32767)
                kk_v[j][sl] = jnp.bitwise_or(jnp.left_shift(t15, 17),
                                             kk_v[j][sl])
                return carry

            plsc.parallel_loop(0, _C // _L, unroll=2,
                               carry=jnp.int32(0))(pstep)
            wr[j] = (
                pltpu.async_copy(kk_v[j],
                                 scr_hbm.at[pl.ds(c * _K + offs[t2], _C)],
                                 so[j]),
                pltpu.async_copy(u_v[j], prob_s.at[pl.ds(offs[t2], _C)],
                                 sg0),
            )
            if t2 + 1 < 4:
                rd = rd_next
        for w2 in wr:
            if w2 is not None:
                w2[0].wait()
                w2[1].wait()
        plsc.subcore_barrier()
    in_copies(0, 0)
    in_copies(1, 1)
    with jax.named_scope("pack_pull"):
        _H = _K // 2
        p1 = pltpu.async_copy(scr_hbm.at[pl.ds(c * _K, _H)],
                              packed_t.at[pl.ds(0, _H)], sp0)
        p2 = pltpu.async_copy(scr_hbm.at[pl.ds(c * _K + _H, _H)],
                              packed_t.at[pl.ds(_H, _H)], sp1)
        p1.wait()
        p2.wait()

    def chunk_pair(i2, carry):
      for j in (0, 1):
        i = i2 * 2 + j
        off = base + i * _C

        @pl.when(i2 >= 1)
        def _wait_prev(i=i, j=j):
            pltpu.make_async_copy(
                o_v[j],
                out_hbm.at[pl.ds(rbase + (i - 2) * _RC, _RC), :],
                so[j]).wait()

        pltpu.make_async_copy(
            kk_hbm.at[pl.ds(off, _C)], kk_v[j], sk[j]).wait()
        pltpu.make_async_copy(
            u_hbm.at[pl.ds(off, _C)], u_v[j], su[j]).wait()

        with jax.named_scope("sel"):
            def step(t, carry, j=j):
                row, col, cnt, lo, hi = carry
                sl = pl.ds(t * _L, _L)
                kkv = kk_v[j][sl]
                w = plsc.bitcast(plsc.load_gather(packed_t, [kkv]),
                                 jnp.uint32)
                k15 = (u_v[j][sl] * 32768.0).astype(jnp.int32)
                t15, av = _unpack(w)
                val = jnp.where(k15 < t15, kkv, av)
                plsc.store_scatter(o_v[j], [row, col], val)
                pc = plsc.all_reduce_population_count(_ambig(k15, t15))
                cnt = cnt + pc
                lo = jnp.minimum(lo, jnp.where(pc > 0, t, _C))
                hi = jnp.maximum(hi, jnp.where(pc > 0, t + 1, 0))
                col = col + _L
                wrap = col >= _NS
                col = jnp.where(wrap, col - _NS, col)
                row = row + wrap.astype(jnp.int32)
                return row, col, cnt, lo, hi

            init = (jnp.zeros((_L,), jnp.int32), lax.iota(jnp.int32, _L),
                    jnp.zeros((_L,), jnp.int32),
                    jnp.full((_L,), _C, jnp.int32),
                    jnp.zeros((_L,), jnp.int32))
            _, _, cnt_v, lo_v, hi_v = plsc.parallel_loop(
                0, _C // _L, unroll=4, carry=init)(step)
            cnt = jnp.max(cnt_v)

        @pl.when(cnt > 0)
        def _fixup(i=i, j=j, lo_v=lo_v, hi_v=hi_v):
            with jax.named_scope("fix"):
                def cap(t, cc, j=j):
                    sl = pl.ds(t * _L, _L)
                    kkv = kk_v[j][sl]
                    w = plsc.bitcast(plsc.load_gather(packed_t, [kkv]),
                                     jnp.uint32)
                    k15 = (u_v[j][sl] * 32768.0).astype(jnp.int32)
                    t15, _ = _unpack(w)
                    m = _ambig(k15, t15)
                    dst = pl.ds(jnp.minimum(cc, _FCAP - _L), _L)
                    plsc.store_compressed(
                        fpos.at[dst], t * _L + lax.iota(jnp.int32, _L),
                        mask=m)
                    return cc + jnp.max(
                        plsc.all_reduce_population_count(m))

                nfix = lax.fori_loop(jnp.max(lo_v), jnp.max(hi_v), cap,
                                     jnp.int32(0))
                nfix = jnp.minimum(nfix, _FCAP - _L)

                def istep(t, carry, j=j):
                    sl = pl.ds(t * _L, _L)
                    fkk[sl] = plsc.load_gather(kk_v[j], [fpos[sl]])
                    return carry

                lax.fori_loop(0, _FCAP // _L, istep, 0)
                pltpu.async_copy(prob_s.at[fkk], fp, sg0).wait()

                def fstep(t, carry, j=j):
                    sl = pl.ds(t * _L, _L)
                    pos = fpos[sl]
                    kkv = fkk[sl]
                    uv = plsc.load_gather(u_v[j], [pos])
                    b = uv < fp[sl]
                    w = plsc.bitcast(plsc.load_gather(packed_t, [kkv]),
                                     jnp.uint32)
                    _, av = _unpack(w)
                    val = jnp.where(b, kkv, av)
                    row = (pos * 5243) >> 18
                    colx = pos - row * _NS
                    lane = t * _L + lax.iota(jnp.int32, _L)
                    plsc.store_scatter(o_v[j], [row, colx], val,
                                       mask=lane < nfix)
                    return carry

                lax.fori_loop(0, _FCAP // _L, fstep, 0)

        pltpu.async_copy(
            o_v[j], out_hbm.at[pl.ds(rbase + i * _RC, _RC), :], so[j])

        @pl.when(i2 < (_NCHUNK // 2) - 1)
        def _next_in(i=i, j=j):
            in_copies(i + 2, j)
      return carry

    lax.fori_loop(0, _NCHUNK // 2, chunk_pair, 0)

    pltpu.make_async_copy(
        o_v[0], out_hbm.at[pl.ds(rbase + 14 * _RC, _RC), :], so[0]).wait()
    pltpu.make_async_copy(
        o_v[1], out_hbm.at[pl.ds(rbase + 15 * _RC, _RC), :], so[1]).wait()


@jax.jit
def _sample(prob, alias, kk, u):
    mesh = plsc.VectorSubcoreMesh(core_axis_name="c", subcore_axis_name="s")
    f = pl.kernel(
        _body,
        mesh=mesh,
        compiler_params=pltpu.CompilerParams(needs_layout_passes=False),
        out_type=(jax.ShapeDtypeStruct((_B, _OW), jnp.int32),
                  jax.ShapeDtypeStruct((2 * _K,), jnp.int32)),
        scratch_types=[
            pltpu.VMEM((_K,), jnp.int32),
            pltpu.VMEM_SHARED((_K,), jnp.float32),
            pltpu.VMEM((_C,), jnp.int32),
            pltpu.VMEM((_C,), jnp.int32),
            pltpu.VMEM((_C,), jnp.float32),
            pltpu.VMEM((_C,), jnp.float32),
            pltpu.VMEM((_RC, _OW), jnp.int32),
            pltpu.VMEM((_RC, _OW), jnp.int32),
            pltpu.VMEM((_FCAP,), jnp.int32),
            pltpu.VMEM((_FCAP,), jnp.int32),
            pltpu.VMEM((_FCAP,), jnp.float32),
        ] + [pltpu.SemaphoreType.DMA] * 9,
    )
    out, _ = f(prob, alias, kk, u)
    return out


def kernel(prob, alias, kk, u):
    return _sample(prob, alias, kk, u)[:, :_NS]
